# baseline (speedup 1.0000x reference)
import jax
import jax.numpy as jnp
from jax.experimental import pallas as pl
from jax.experimental.pallas import tpu as pltpu

B = 4
S = 2048
DIM = 1024
NA = 3
CH = 512

_ACT = 0
_RES = 1
_PEND = 2


def _fused_kernel(h_ref, wsel_ref, bsel_ref, w1_hbm, w2_hbm, o_ref,
                  wf_ref, w1b_ref, w2b_ref, xb_ref, hb_ref,
                  state_ref, sem1, sem2):
    b = pl.program_id(0)
    prev_act = jnp.where(b == 0, 0, state_ref[_ACT])
    pending = jnp.where(b == 0, 0, state_ref[_PEND])

    def _route():
        hv = h_ref[0]
        pooled = jnp.sum(hv, axis=0) * (1.0 / S)
        best = jnp.sum(wsel_ref[0] * pooled) + bsel_ref[0]
        best_i = jnp.int32(0)
        for j in range(1, NA + 1):
            lj = jnp.sum(wsel_ref[j] * pooled) + bsel_ref[j]
            take = lj > best
            best_i = jnp.where(take, jnp.int32(j), best_i)
            best = jnp.maximum(lj, best)
        active = best_i >= 1
        sel = best_i - 1

        resident = jnp.where(b == 0, jnp.int32(-1), state_ref[_RES])
        need = jnp.logical_and(active, sel != resident)

        @pl.when(jnp.logical_and(b < B, need))
        def _fetch():
            pltpu.make_async_copy(w1_hbm.at[sel], wf_ref, sem1).start()

        @pl.when(b < B)
        def _commit():
            state_ref[_ACT] = active.astype(jnp.int32)
            state_ref[_RES] = jnp.where(active, sel, resident)
            state_ref[_PEND] = need.astype(jnp.int32)

        xb_ref[jax.lax.rem(b, 2)] = hv.astype(jnp.bfloat16)

    mlp_pred = jnp.logical_and(b > 0, prev_act == 1)

    @pl.when(mlp_pred)
    def _mlp():
        @pl.when(pending == 1)
        def _land_w1():
            pltpu.make_async_copy(w1_hbm.at[0], wf_ref, sem1).wait()
            w1b_ref[...] = wf_ref[...].astype(jnp.bfloat16)
            pltpu.make_async_copy(w2_hbm.at[state_ref[_RES]], wf_ref,
                                  sem2).start()

        slot = jax.lax.rem(b - 1, 2)
        for c in range(S // CH):
            rows = pl.ds(c * CH, CH)
            x = xb_ref[slot, rows, :]
            hid = jnp.dot(x, w1b_ref[...], preferred_element_type=jnp.float32)
            hid = 0.5 * hid * (1.0 + jax.lax.erf(hid * 0.7071067811865476))
            hb_ref[rows, :] = hid.astype(jnp.bfloat16)

        @pl.when(pending == 1)
        def _land_w2():
            pltpu.make_async_copy(w2_hbm.at[0], wf_ref, sem2).wait()
            w2b_ref[...] = wf_ref[...].astype(jnp.bfloat16)

        for c in range(S // CH):
            rows = pl.ds(c * CH, CH)
            delta = jnp.dot(hb_ref[rows, :], w2b_ref[...],
                            preferred_element_type=jnp.float32)
            o_ref[0, rows, :] = (
                xb_ref[slot, rows, :].astype(jnp.float32) + delta)
        _route()

    @pl.when(jnp.logical_not(mlp_pred))
    def _copy_or_first():
        @pl.when(jnp.logical_and(b > 0, prev_act == 0))
        def _copy():
            o_ref[0] = xb_ref[jax.lax.rem(b - 1, 2)].astype(jnp.float32)
        _route()


@jax.jit
def kernel(h, W_sel, b_sel, W1, W2):
    out = pl.pallas_call(
        _fused_kernel,
        grid=(B + 1,),
        in_specs=[
            pl.BlockSpec((1, S, DIM), lambda b: (jnp.minimum(b, B - 1), 0, 0)),
            pl.BlockSpec((NA + 1, DIM), lambda b: (0, 0)),
            pl.BlockSpec(memory_space=pltpu.SMEM),
            pl.BlockSpec(memory_space=pltpu.HBM),
            pl.BlockSpec(memory_space=pltpu.HBM),
        ],
        out_specs=pl.BlockSpec((1, S, DIM), lambda b: (jnp.maximum(b - 1, 0), 0, 0)),
        out_shape=jax.ShapeDtypeStruct((B, S, DIM), jnp.float32),
        scratch_shapes=[
            pltpu.VMEM((DIM, DIM), jnp.float32),
            pltpu.VMEM((DIM, DIM), jnp.bfloat16),
            pltpu.VMEM((DIM, DIM), jnp.bfloat16),
            pltpu.VMEM((2, S, DIM), jnp.bfloat16),
            pltpu.VMEM((S, DIM), jnp.bfloat16),
            pltpu.SMEM((3,), jnp.int32),
            pltpu.SemaphoreType.DMA,
            pltpu.SemaphoreType.DMA,
        ],
        compiler_params=pltpu.CompilerParams(
            dimension_semantics=(pltpu.ARBITRARY,),
        ),
    )(h, W_sel, b_sel, W1, W2)
    return out


# speedup vs baseline: 1.0913x; 1.0913x over previous
import jax
import jax.numpy as jnp
from jax.experimental import pallas as pl
from jax.experimental.pallas import tpu as pltpu

B = 4
S = 2048
DIM = 1024
NA = 3
CH = 512

_ACT = 0
_RES = 1
_PEND = 2


def _fused_kernel(h_ref, wsel_ref, bsel_ref, w1_hbm, w2_hbm, o_ref,
                  w1f_ref, w2f_ref, w1b_ref, w2b_ref, xb_ref, state_ref,
                  sem1, sem2):
    b = pl.program_id(0)
    prev_act = jnp.where(b == 0, 0, state_ref[_ACT])
    pending = jnp.where(b == 0, 0, state_ref[_PEND])

    def _route_tail(pooled):
        best = jnp.sum(wsel_ref[0] * pooled) + bsel_ref[0]
        best_i = jnp.int32(0)
        for j in range(1, NA + 1):
            lj = jnp.sum(wsel_ref[j] * pooled) + bsel_ref[j]
            take = lj > best
            best_i = jnp.where(take, jnp.int32(j), best_i)
            best = jnp.maximum(lj, best)
        active = best_i >= 1
        sel = best_i - 1

        resident = jnp.where(b == 0, jnp.int32(-1), state_ref[_RES])
        need = jnp.logical_and(active, sel != resident)

        @pl.when(jnp.logical_and(b < B, need))
        def _fetch():
            pltpu.make_async_copy(w1_hbm.at[sel], w1f_ref, sem1).start()
            pltpu.make_async_copy(w2_hbm.at[sel], w2f_ref, sem2).start()

        @pl.when(b < B)
        def _commit():
            state_ref[_ACT] = active.astype(jnp.int32)
            state_ref[_RES] = jnp.where(active, sel, resident)
            state_ref[_PEND] = need.astype(jnp.int32)

        xb_ref[jax.lax.rem(b, 2)] = h_ref[0].astype(jnp.bfloat16)

    mlp_pred = jnp.logical_and(b > 0, prev_act == 1)

    @pl.when(mlp_pred)
    def _mlp():
        @pl.when(pending == 1)
        def _land_weights():
            pltpu.make_async_copy(w1_hbm.at[0], w1f_ref, sem1).wait()
            pltpu.make_async_copy(w2_hbm.at[0], w2f_ref, sem2).wait()
            w1b_ref[...] = w1f_ref[...].astype(jnp.bfloat16)
            w2b_ref[...] = w2f_ref[...].astype(jnp.bfloat16)

        slot = jax.lax.rem(b - 1, 2)
        acc = None
        for c in range(S // CH):
            rows = pl.ds(c * CH, CH)
            x = xb_ref[slot, rows, :]
            hid = jnp.dot(x, w1b_ref[...], preferred_element_type=jnp.float32)
            hid = 0.5 * hid * (1.0 + jax.lax.erf(hid * 0.7071067811865476))
            delta = jnp.dot(hid.astype(jnp.bfloat16), w2b_ref[...],
                            preferred_element_type=jnp.float32)
            o_ref[0, rows, :] = x.astype(jnp.float32) + delta
            part = jnp.sum(h_ref[0, rows, :], axis=0)
            acc = part if acc is None else acc + part
        _route_tail(acc * (1.0 / S))

    @pl.when(jnp.logical_not(mlp_pred))
    def _copy_or_first():
        @pl.when(jnp.logical_and(b > 0, prev_act == 0))
        def _copy():
            o_ref[0] = xb_ref[jax.lax.rem(b - 1, 2)].astype(jnp.float32)
        _route_tail(jnp.sum(h_ref[0], axis=0) * (1.0 / S))


@jax.jit
def kernel(h, W_sel, b_sel, W1, W2):
    out = pl.pallas_call(
        _fused_kernel,
        grid=(B + 1,),
        in_specs=[
            pl.BlockSpec((1, S, DIM), lambda b: (jnp.minimum(b, B - 1), 0, 0)),
            pl.BlockSpec((NA + 1, DIM), lambda b: (0, 0)),
            pl.BlockSpec(memory_space=pltpu.SMEM),
            pl.BlockSpec(memory_space=pltpu.HBM),
            pl.BlockSpec(memory_space=pltpu.HBM),
        ],
        out_specs=pl.BlockSpec((1, S, DIM), lambda b: (jnp.maximum(b - 1, 0), 0, 0)),
        out_shape=jax.ShapeDtypeStruct((B, S, DIM), jnp.float32),
        scratch_shapes=[
            pltpu.VMEM((DIM, DIM), jnp.float32),
            pltpu.VMEM((DIM, DIM), jnp.float32),
            pltpu.VMEM((DIM, DIM), jnp.bfloat16),
            pltpu.VMEM((DIM, DIM), jnp.bfloat16),
            pltpu.VMEM((2, S, DIM), jnp.bfloat16),
            pltpu.SMEM((3,), jnp.int32),
            pltpu.SemaphoreType.DMA,
            pltpu.SemaphoreType.DMA,
        ],
        compiler_params=pltpu.CompilerParams(
            dimension_semantics=(pltpu.ARBITRARY,),
        ),
    )(h, W_sel, b_sel, W1, W2)
    return out


# speedup vs baseline: 1.1146x; 1.0213x over previous
import jax
import jax.numpy as jnp
from jax.experimental import pallas as pl
from jax.experimental.pallas import tpu as pltpu

B = 4
S = 2048
DIM = 1024
NA = 3
CH = 256

_ACT = 0
_RES = 1
_PEND = 2


def _fused_kernel(h_ref, wsel_ref, bsel_ref, w1_hbm, w2_hbm, o_ref,
                  w1f_ref, w2f_ref, w1b_ref, w2b_ref, xb_ref, state_ref,
                  sem1, sem2):
    b = pl.program_id(0)
    prev_act = jnp.where(b == 0, 0, state_ref[_ACT])
    pending = jnp.where(b == 0, 0, state_ref[_PEND])

    def _route():
        hv = h_ref[0]
        pooled = jnp.sum(hv, axis=0) * (1.0 / S)
        best = jnp.sum(wsel_ref[0] * pooled) + bsel_ref[0]
        best_i = jnp.int32(0)
        for j in range(1, NA + 1):
            lj = jnp.sum(wsel_ref[j] * pooled) + bsel_ref[j]
            take = lj > best
            best_i = jnp.where(take, jnp.int32(j), best_i)
            best = jnp.maximum(lj, best)
        active = best_i >= 1
        sel = best_i - 1

        resident = jnp.where(b == 0, jnp.int32(-1), state_ref[_RES])
        need = jnp.logical_and(active, sel != resident)

        @pl.when(jnp.logical_and(b < B, need))
        def _fetch():
            pltpu.make_async_copy(w1_hbm.at[sel], w1f_ref, sem1).start()
            pltpu.make_async_copy(w2_hbm.at[sel], w2f_ref, sem2).start()

        @pl.when(b < B)
        def _commit():
            state_ref[_ACT] = active.astype(jnp.int32)
            state_ref[_RES] = jnp.where(active, sel, resident)
            state_ref[_PEND] = need.astype(jnp.int32)

        xb_ref[jax.lax.rem(b, 2)] = hv.astype(jnp.bfloat16)

    mlp_pred = jnp.logical_and(b > 0, prev_act == 1)

    @pl.when(mlp_pred)
    def _mlp():
        @pl.when(pending == 1)
        def _land_weights():
            pltpu.make_async_copy(w1_hbm.at[0], w1f_ref, sem1).wait()
            pltpu.make_async_copy(w2_hbm.at[0], w2f_ref, sem2).wait()
            w1b_ref[...] = w1f_ref[...].astype(jnp.bfloat16)
            w2b_ref[...] = w2f_ref[...].astype(jnp.bfloat16)

        slot = jax.lax.rem(b - 1, 2)
        for c in range(S // CH):
            rows = pl.ds(c * CH, CH)
            x = xb_ref[slot, rows, :]
            hid = jnp.dot(x, w1b_ref[...], preferred_element_type=jnp.float32)
            hid = 0.5 * hid * (1.0 + jax.lax.erf(hid * 0.7071067811865476))
            delta = jnp.dot(hid.astype(jnp.bfloat16), w2b_ref[...],
                            preferred_element_type=jnp.float32)
            o_ref[0, rows, :] = x.astype(jnp.float32) + delta
        _route()

    @pl.when(jnp.logical_not(mlp_pred))
    def _copy_or_first():
        @pl.when(jnp.logical_and(b > 0, prev_act == 0))
        def _copy():
            o_ref[0] = xb_ref[jax.lax.rem(b - 1, 2)].astype(jnp.float32)
        _route()


@jax.jit
def kernel(h, W_sel, b_sel, W1, W2):
    out = pl.pallas_call(
        _fused_kernel,
        grid=(B + 1,),
        in_specs=[
            pl.BlockSpec((1, S, DIM), lambda b: (jnp.minimum(b, B - 1), 0, 0)),
            pl.BlockSpec((NA + 1, DIM), lambda b: (0, 0)),
            pl.BlockSpec(memory_space=pltpu.SMEM),
            pl.BlockSpec(memory_space=pltpu.HBM),
            pl.BlockSpec(memory_space=pltpu.HBM),
        ],
        out_specs=pl.BlockSpec((1, S, DIM), lambda b: (jnp.maximum(b - 1, 0), 0, 0)),
        out_shape=jax.ShapeDtypeStruct((B, S, DIM), jnp.float32),
        scratch_shapes=[
            pltpu.VMEM((DIM, DIM), jnp.float32),
            pltpu.VMEM((DIM, DIM), jnp.float32),
            pltpu.VMEM((DIM, DIM), jnp.bfloat16),
            pltpu.VMEM((DIM, DIM), jnp.bfloat16),
            pltpu.VMEM((2, S, DIM), jnp.bfloat16),
            pltpu.SMEM((3,), jnp.int32),
            pltpu.SemaphoreType.DMA,
            pltpu.SemaphoreType.DMA,
        ],
        compiler_params=pltpu.CompilerParams(
            dimension_semantics=(pltpu.ARBITRARY,),
        ),
    )(h, W_sel, b_sel, W1, W2)
    return out


# speedup vs baseline: 1.1196x; 1.0046x over previous
import jax
import jax.numpy as jnp
from jax.experimental import pallas as pl
from jax.experimental.pallas import tpu as pltpu

B = 4
S = 2048
DIM = 1024
NA = 3
CH = 512

_ACT = 0
_RES = 1
_PEND = 2


def _fused_kernel(h_ref, wsel_ref, bsel_ref, w1_hbm, w2_hbm, o_ref,
                  w1f_ref, w2f_ref, w1b_ref, w2b_ref, xb_ref, state_ref,
                  sem1, sem2):
    b = pl.program_id(0)
    prev_act = jnp.where(b == 0, 0, state_ref[_ACT])
    pending = jnp.where(b == 0, 0, state_ref[_PEND])

    def _route():
        hv = h_ref[0]
        pooled = jnp.sum(hv, axis=0) * (1.0 / S)
        best = jnp.sum(wsel_ref[0] * pooled) + bsel_ref[0]
        best_i = jnp.int32(0)
        for j in range(1, NA + 1):
            lj = jnp.sum(wsel_ref[j] * pooled) + bsel_ref[j]
            take = lj > best
            best_i = jnp.where(take, jnp.int32(j), best_i)
            best = jnp.maximum(lj, best)
        active = best_i >= 1
        sel = best_i - 1

        resident = jnp.where(b == 0, jnp.int32(-1), state_ref[_RES])
        need = jnp.logical_and(active, sel != resident)

        @pl.when(jnp.logical_and(b < B, need))
        def _fetch():
            pltpu.make_async_copy(w1_hbm.at[sel], w1f_ref, sem1).start()
            pltpu.make_async_copy(w2_hbm.at[sel], w2f_ref, sem2).start()

        @pl.when(b < B)
        def _commit():
            state_ref[_ACT] = active.astype(jnp.int32)
            state_ref[_RES] = jnp.where(active, sel, resident)
            state_ref[_PEND] = need.astype(jnp.int32)

        xb_ref[jax.lax.rem(b, 2)] = hv.astype(jnp.bfloat16)

    mlp_pred = jnp.logical_and(b > 0, prev_act == 1)

    @pl.when(mlp_pred)
    def _mlp():
        @pl.when(pending == 1)
        def _land_weights():
            pltpu.make_async_copy(w1_hbm.at[0], w1f_ref, sem1).wait()
            pltpu.make_async_copy(w2_hbm.at[0], w2f_ref, sem2).wait()
            w1b_ref[...] = w1f_ref[...].astype(jnp.bfloat16)
            w2b_ref[...] = w2f_ref[...].astype(jnp.bfloat16)

        slot = jax.lax.rem(b - 1, 2)
        for c in range(S // CH):
            rows = pl.ds(c * CH, CH)
            x = xb_ref[slot, rows, :]
            hid = jnp.dot(x, w1b_ref[...], preferred_element_type=jnp.float32)
            hb = hid.astype(jnp.bfloat16)
            one = jnp.bfloat16(1.0)
            halfv = jnp.bfloat16(0.5)
            cinv = jnp.bfloat16(0.7071067811865476)
            hb = halfv * hb * (one + jax.lax.erf(hb * cinv))
            delta = jnp.dot(hb, w2b_ref[...],
                            preferred_element_type=jnp.float32)
            o_ref[0, rows, :] = x.astype(jnp.float32) + delta
        _route()

    @pl.when(jnp.logical_not(mlp_pred))
    def _copy_or_first():
        @pl.when(jnp.logical_and(b > 0, prev_act == 0))
        def _copy():
            o_ref[0] = xb_ref[jax.lax.rem(b - 1, 2)].astype(jnp.float32)
        _route()


@jax.jit
def kernel(h, W_sel, b_sel, W1, W2):
    out = pl.pallas_call(
        _fused_kernel,
        grid=(B + 1,),
        in_specs=[
            pl.BlockSpec((1, S, DIM), lambda b: (jnp.minimum(b, B - 1), 0, 0)),
            pl.BlockSpec((NA + 1, DIM), lambda b: (0, 0)),
            pl.BlockSpec(memory_space=pltpu.SMEM),
            pl.BlockSpec(memory_space=pltpu.HBM),
            pl.BlockSpec(memory_space=pltpu.HBM),
        ],
        out_specs=pl.BlockSpec((1, S, DIM), lambda b: (jnp.maximum(b - 1, 0), 0, 0)),
        out_shape=jax.ShapeDtypeStruct((B, S, DIM), jnp.float32),
        scratch_shapes=[
            pltpu.VMEM((DIM, DIM), jnp.float32),
            pltpu.VMEM((DIM, DIM), jnp.float32),
            pltpu.VMEM((DIM, DIM), jnp.bfloat16),
            pltpu.VMEM((DIM, DIM), jnp.bfloat16),
            pltpu.VMEM((2, S, DIM), jnp.bfloat16),
            pltpu.SMEM((3,), jnp.int32),
            pltpu.SemaphoreType.DMA,
            pltpu.SemaphoreType.DMA,
        ],
        compiler_params=pltpu.CompilerParams(
            dimension_semantics=(pltpu.ARBITRARY,),
        ),
    )(h, W_sel, b_sel, W1, W2)
    return out


# speedup vs baseline: 1.1221x; 1.0022x over previous
import jax
import jax.numpy as jnp
from jax.experimental import pallas as pl
from jax.experimental.pallas import tpu as pltpu

B = 4
S = 2048
DIM = 1024
NA = 3
CH = 512

_ACT = 0
_RES = 1
_PEND = 2


def _fused_kernel(h_ref, wsel_ref, bsel_ref, w1_hbm, w2_hbm, o_ref,
                  w1f_ref, w2f_ref, w1b_ref, w2b_ref, xb_ref, state_ref,
                  sem1, sem2):
    b = pl.program_id(0)
    prev_act = jnp.where(b == 0, 0, state_ref[_ACT])
    pending = jnp.where(b == 0, 0, state_ref[_PEND])

    def _route():
        hv = h_ref[0]
        pooled = jnp.sum(hv, axis=0) * (1.0 / S)
        best = jnp.sum(wsel_ref[0] * pooled) + bsel_ref[0]
        best_i = jnp.int32(0)
        for j in range(1, NA + 1):
            lj = jnp.sum(wsel_ref[j] * pooled) + bsel_ref[j]
            take = lj > best
            best_i = jnp.where(take, jnp.int32(j), best_i)
            best = jnp.maximum(lj, best)
        active = best_i >= 1
        sel = best_i - 1

        resident = jnp.where(b == 0, jnp.int32(-1), state_ref[_RES])
        need = jnp.logical_and(active, sel != resident)

        @pl.when(jnp.logical_and(b < B, need))
        def _fetch():
            pltpu.make_async_copy(w1_hbm.at[sel], w1f_ref, sem1).start()
            pltpu.make_async_copy(w2_hbm.at[sel], w2f_ref, sem2).start()

        @pl.when(b < B)
        def _commit():
            state_ref[_ACT] = active.astype(jnp.int32)
            state_ref[_RES] = jnp.where(active, sel, resident)
            state_ref[_PEND] = need.astype(jnp.int32)

        xb_ref[jax.lax.rem(b, 2)] = hv.astype(jnp.bfloat16)

    mlp_pred = jnp.logical_and(b > 0, prev_act == 1)

    @pl.when(mlp_pred)
    def _mlp():
        @pl.when(pending == 1)
        def _land_weights():
            pltpu.make_async_copy(w1_hbm.at[0], w1f_ref, sem1).wait()
            pltpu.make_async_copy(w2_hbm.at[0], w2f_ref, sem2).wait()
            w1b_ref[...] = w1f_ref[...].astype(jnp.bfloat16)
            w2b_ref[...] = w2f_ref[...].astype(jnp.bfloat16)

        slot = jax.lax.rem(b - 1, 2)
        for c in range(S // CH):
            rows = pl.ds(c * CH, CH)
            x = xb_ref[slot, rows, :]
            hid = jnp.dot(x, w1b_ref[...], preferred_element_type=jnp.float32)
            hid = 0.5 * hid * (1.0 + jax.lax.erf(hid * 0.7071067811865476))
            delta = jnp.dot(hid.astype(jnp.bfloat16), w2b_ref[...],
                            preferred_element_type=jnp.float32)
            o_ref[0, rows, :] = x.astype(jnp.float32) + delta
        _route()

    @pl.when(jnp.logical_not(mlp_pred))
    def _copy_or_first():
        @pl.when(jnp.logical_and(b > 0, prev_act == 0))
        def _copy():
            o_ref[0] = xb_ref[jax.lax.rem(b - 1, 2)].astype(jnp.float32)
        _route()


@jax.jit
def kernel(h, W_sel, b_sel, W1, W2):
    out = pl.pallas_call(
        _fused_kernel,
        grid=(B + 1,),
        in_specs=[
            pl.BlockSpec((1, S, DIM), lambda b: (jnp.minimum(b, B - 1), 0, 0)),
            pl.BlockSpec((NA + 1, DIM), lambda b: (0, 0)),
            pl.BlockSpec(memory_space=pltpu.SMEM),
            pl.BlockSpec(memory_space=pltpu.HBM),
            pl.BlockSpec(memory_space=pltpu.HBM),
        ],
        out_specs=pl.BlockSpec((1, S, DIM), lambda b: (jnp.maximum(b - 1, 0), 0, 0)),
        out_shape=jax.ShapeDtypeStruct((B, S, DIM), jnp.float32),
        scratch_shapes=[
            pltpu.VMEM((DIM, DIM), jnp.float32),
            pltpu.VMEM((DIM, DIM), jnp.float32),
            pltpu.VMEM((DIM, DIM), jnp.bfloat16),
            pltpu.VMEM((DIM, DIM), jnp.bfloat16),
            pltpu.VMEM((2, S, DIM), jnp.bfloat16),
            pltpu.SMEM((3,), jnp.int32),
            pltpu.SemaphoreType.DMA,
            pltpu.SemaphoreType.DMA,
        ],
        compiler_params=pltpu.CompilerParams(
            dimension_semantics=(pltpu.ARBITRARY,),
        ),
    )(h, W_sel, b_sel, W1, W2)
    return out
